# baseline (speedup 1.0000x reference)
import functools

import jax
import jax.numpy as jnp
from jax.experimental import pallas as pl
from jax.experimental.pallas import tpu as pltpu


def _mole_kernel(x_ref, a_ref, wcat_ref, b_ref, out_ref,
                 xh_scratch, *, D_IN, ER, R):
    x_tile = x_ref[...]
    xbf = x_tile.astype(jnp.bfloat16)
    xh_scratch[:, :D_IN] = xbf
    he = jax.lax.dot_general(
        xbf, a_ref[...], (((1,), (1,)), ((), ())),
        preferred_element_type=jnp.float32)
    idx = jnp.argmax(he[:, ER:], axis=1)
    h_all = he[:, :ER]
    col = jax.lax.broadcasted_iota(jnp.int32, h_all.shape, 1)
    mask = (col // R) == idx[:, None]
    xh_scratch[:, D_IN:] = jnp.where(mask, h_all, 0.0).astype(jnp.bfloat16)
    out_ref[...] = jax.lax.dot_general(
        xh_scratch[...], wcat_ref[...], (((1,), (1,)), ((), ())),
        preferred_element_type=jnp.float32) + b_ref[...]


@jax.jit
def kernel(x, W_base, b_base, W_router, A, B):
    Bsz, S, D_IN = x.shape
    D_OUT = W_base.shape[0]
    E, R, _ = A.shape
    ER = E * R
    ALPHA = 16.0
    SCALING = ALPHA / R
    T = Bsz * S

    Tt = min(256, T)
    n_t = T // Tt

    x2 = x.reshape(T, D_IN)
    A_ext = jnp.concatenate(
        [A.reshape(ER, D_IN), W_router], axis=0).astype(jnp.bfloat16)
    B_rT = B.transpose(1, 0, 2).reshape(D_OUT, ER) * SCALING
    W_cat = jnp.concatenate([W_base, B_rT], axis=1).astype(jnp.bfloat16)
    b2 = b_base.reshape(1, D_OUT)

    out = pl.pallas_call(
        functools.partial(_mole_kernel, D_IN=D_IN, ER=ER, R=R),
        grid=(n_t,),
        in_specs=[
            pl.BlockSpec((Tt, D_IN), lambda t: (t, 0)),
            pl.BlockSpec((ER + E, D_IN), lambda t: (0, 0)),
            pl.BlockSpec((D_OUT, D_IN + ER), lambda t: (0, 0)),
            pl.BlockSpec((1, D_OUT), lambda t: (0, 0)),
        ],
        out_specs=pl.BlockSpec((Tt, D_OUT), lambda t: (t, 0)),
        out_shape=jax.ShapeDtypeStruct((T, D_OUT), jnp.float32),
        scratch_shapes=[
            pltpu.VMEM((Tt, D_IN + ER), jnp.bfloat16),
        ],
        compiler_params=pltpu.CompilerParams(
            dimension_semantics=("parallel",),
            vmem_limit_bytes=100 * 1024 * 1024,
        ),
    )(x2, A_ext, W_cat, b2)

    return out.reshape(Bsz, S, D_OUT)


# speedup vs baseline: 1.0071x; 1.0071x over previous
import functools

import jax
import jax.numpy as jnp
from jax.experimental import pallas as pl
from jax.experimental.pallas import tpu as pltpu


def _mole_kernel(x_ref, wr_ref, a_ref, wcat_ref, b_ref, out_ref,
                 xh_scratch, *, D_IN, R, SCALING):
    x_tile = x_ref[...]
    xh_scratch[:, :D_IN] = x_tile.astype(jnp.bfloat16)
    logits = jax.lax.dot_general(
        x_tile, wr_ref[...], (((1,), (1,)), ((), ())),
        preferred_element_type=jnp.float32)
    idx = jnp.argmax(logits, axis=1)
    h_all = jax.lax.dot_general(
        xh_scratch[:, :D_IN], a_ref[...], (((1,), (1,)), ((), ())),
        preferred_element_type=jnp.float32)
    col = jax.lax.broadcasted_iota(jnp.int32, h_all.shape, 1)
    mask = (col // R) == idx[:, None]
    xh_scratch[:, D_IN:] = jnp.where(mask, h_all * SCALING, 0.0).astype(
        jnp.bfloat16)
    out_ref[...] = jax.lax.dot_general(
        xh_scratch[...], wcat_ref[...], (((1,), (1,)), ((), ())),
        preferred_element_type=jnp.float32) + b_ref[...]


@jax.jit
def kernel(x, W_base, b_base, W_router, A, B):
    Bsz, S, D_IN = x.shape
    D_OUT = W_base.shape[0]
    E, R, _ = A.shape
    ER = E * R
    ALPHA = 16.0
    SCALING = ALPHA / R
    T = Bsz * S

    Tt = min(256, T)
    n_t = T // Tt

    x2 = x.reshape(T, D_IN)
    A_all = A.reshape(ER, D_IN).astype(jnp.bfloat16)
    B_rT = B.transpose(1, 0, 2).reshape(D_OUT, ER)
    W_cat = jnp.concatenate([W_base, B_rT], axis=1).astype(jnp.bfloat16)
    b2 = b_base.reshape(1, D_OUT)

    out = pl.pallas_call(
        functools.partial(_mole_kernel, D_IN=D_IN, R=R, SCALING=SCALING),
        grid=(n_t,),
        in_specs=[
            pl.BlockSpec((Tt, D_IN), lambda t: (t, 0)),
            pl.BlockSpec((E, D_IN), lambda t: (0, 0)),
            pl.BlockSpec((ER, D_IN), lambda t: (0, 0)),
            pl.BlockSpec((D_OUT, D_IN + ER), lambda t: (0, 0)),
            pl.BlockSpec((1, D_OUT), lambda t: (0, 0)),
        ],
        out_specs=pl.BlockSpec((Tt, D_OUT), lambda t: (t, 0)),
        out_shape=jax.ShapeDtypeStruct((T, D_OUT), jnp.float32),
        scratch_shapes=[
            pltpu.VMEM((Tt, D_IN + ER), jnp.bfloat16),
        ],
        compiler_params=pltpu.CompilerParams(
            dimension_semantics=("parallel",),
            vmem_limit_bytes=100 * 1024 * 1024,
        ),
    )(x2, W_router, A_all, W_cat, b2)

    return out.reshape(Bsz, S, D_OUT)
